# baseline (speedup 1.0000x reference)
import functools

import jax
import jax.numpy as jnp
from jax import lax
from jax.experimental import pallas as pl
from jax.experimental.pallas import tpu as pltpu
from jax.experimental.pallas import tpu_sc as plsc

N = 4096
DIM = 16
KSEL = 31
RB = 256
NBISECT = 10

NC = 2
NS = 16
L = 16
NW = NC * NS
RPW = N // NW
SCB = 8
NBATCH = RPW // SCB
NCH = 256
BIG = 1e30


def _d2_body(x_ref, out_ref, thr_ref, cm_ref):
    i = pl.program_id(0)
    X = x_ref[...]
    xb = x_ref[pl.ds(i * RB, RB), :]
    sq = jnp.sum(X * X, axis=1)
    sqb = jnp.sum(xb * xb, axis=1)
    dot = lax.dot_general(xb, X, (((1,), (1,)), ((), ())),
                          preferred_element_type=jnp.float32)
    d2 = jnp.maximum(sqb[:, None] + sq[None, :] - 2.0 * dot, 0.0)
    out_ref[...] = d2

    hi = (jnp.sqrt(sqb) + jnp.sqrt(jnp.max(sq))) ** 2 + 1.0
    lo = jnp.zeros((RB,), jnp.float32)
    for _ in range(NBISECT):
        mid = 0.5 * (lo + hi)
        c = jnp.sum((d2 < mid[:, None]).astype(jnp.int32), axis=1)
        ge = c >= KSEL
        hi = jnp.where(ge, mid, hi)
        lo = jnp.where(ge, lo, mid)
    thr_ref[...] = hi

    cm = d2[:, 0:NCH]
    for k in range(1, N // NCH):
        cm = jnp.minimum(cm, d2[:, k * NCH:(k + 1) * NCH])
    cm_ref[...] = cm


def _pairwise_d2(X):
    return pl.pallas_call(
        _d2_body,
        grid=(N // RB,),
        in_specs=[pl.BlockSpec((N, DIM), lambda i: (0, 0))],
        out_specs=[pl.BlockSpec((RB, N), lambda i: (i, 0)),
                   pl.BlockSpec((RB,), lambda i: (i,)),
                   pl.BlockSpec((RB, NCH), lambda i: (i, 0))],
        out_shape=[jax.ShapeDtypeStruct((N, N), jnp.float32),
                   jax.ShapeDtypeStruct((N,), jnp.float32),
                   jax.ShapeDtypeStruct((N, NCH), jnp.float32)],
    )(X)


def _merge32(A, B, v):
    vs = lax.sort(v)
    lo = jnp.minimum(B, lax.rev(vs, (0,)))
    rlos = lax.rev(lax.sort(lo), (0,))
    A2 = lax.sort(jnp.minimum(A, rlos))
    B2 = lax.sort(jnp.maximum(A, rlos))
    return A2, B2


def _filter_row(rows, cms, r, t, cand_v):
    tv = jnp.broadcast_to(t, (L,))
    lane = lax.iota(jnp.int32, L)
    rvec = jnp.broadcast_to(r, (L,))
    off = 0
    for v16 in range(NCH // L):
        cmv = cms[r, pl.ds(v16 * L, L)]
        chm = cmv < tv
        nset = plsc.all_reduce_population_count(chm)[0]

        def surv(_, st):
            chm, off = st
            g = plsc.all_reduce_ffs(chm)[0]
            chm = chm & (lane != g)
            cidx = (v16 * L + g) + NCH * lane
            vk = plsc.load_gather(rows, [rvec, cidx])
            m2 = vk < tv
            plsc.store_compressed(cand_v.at[pl.ds(off, L)], vk, mask=m2)
            off = off + plsc.all_reduce_population_count(m2)[0]
            return chm, off

        chm, off = lax.fori_loop(0, nset, surv, (chm, off))
    return off


def _select_raw(rows, r):
    v0 = lax.sort(rows[r, pl.ds(0, L)])
    rv1 = lax.rev(lax.sort(rows[r, pl.ds(L, L)]), (0,))
    A = lax.sort(jnp.minimum(v0, rv1))
    B = lax.sort(jnp.maximum(v0, rv1))

    def cb(i, carry):
        return _merge32(carry[0], carry[1], rows[r, pl.ds(i * L, L)])

    A, B = lax.fori_loop(2, N // L, cb, (A, B))
    a15 = jnp.max(A)
    lane = lax.iota(jnp.int32, L)
    b14 = jnp.max(jnp.where(lane < L - 1, B, -1.0))
    return a15, b14


def _select_stats(cand_v, m):
    inf_v = jnp.full((L,), BIG, jnp.float32)
    cand_v[pl.ds(m, L)] = inf_v
    cand_v[pl.ds(m + L, L)] = inf_v
    v0 = lax.sort(cand_v[pl.ds(0, L)])
    rv1 = lax.rev(lax.sort(cand_v[pl.ds(L, L)]), (0,))
    A = lax.sort(jnp.minimum(v0, rv1))
    B = lax.sort(jnp.maximum(v0, rv1))

    def chunk_body(i, carry):
        v = cand_v[pl.ds(i * L, L)]
        return _merge32(carry[0], carry[1], v)

    nch = (m + L - 1) // L
    A, B = lax.fori_loop(2, nch, chunk_body, (A, B))
    a15 = jnp.max(A)
    lane = lax.iota(jnp.int32, L)
    b14 = jnp.max(jnp.where(lane < L - 1, B, -1.0))
    return a15, b14


def _stats_body(d2_hbm, thr_hbm, cm_hbm, a_hbm, b_hbm,
                rows0_v, rows1_v, cm0_v, cm1_v, cand_v, thr_v, a_v, b_v,
                sem0, sem1, semc0, semc1):
    wid = lax.axis_index("s") * NC + lax.axis_index("c")
    base = wid * RPW
    pltpu.sync_copy(thr_hbm.at[pl.ds(base, RPW)], thr_v)

    def copy(bi, buf_ref, sem):
        return pltpu.make_async_copy(
            d2_hbm.at[pl.ds(base + bi * SCB, SCB), :], buf_ref, sem)

    def copyc(bi, buf_ref, sem):
        return pltpu.make_async_copy(
            cm_hbm.at[pl.ds(base + bi * SCB, SCB), :], buf_ref, sem)

    copy(0, rows0_v, sem0).start()
    copyc(0, cm0_v, semc0).start()

    def half(j, half_idx, buf_ref, cm_ref, acc):
        accA, accB = acc
        lane = lax.iota(jnp.int32, L)
        tvec = thr_v[pl.ds(j * L, L)]

        def row_body(r, acc):
            accA, accB = acc
            li = half_idx * SCB + r
            t = jnp.max(jnp.where(lane == li, tvec, -1.0))
            m = _filter_row(buf_ref, cm_ref, r, t, cand_v)
            a15, b14 = lax.cond(m >= KSEL,
                                lambda: _select_stats(cand_v, m),
                                lambda: _select_raw(buf_ref, r))
            accA = jnp.where(lane == li, a15, accA)
            accB = jnp.where(lane == li, b14, accB)
            return accA, accB

        return lax.fori_loop(0, SCB, row_body, (accA, accB))

    def pair_body(j, _):
        zero = jnp.zeros((L,), jnp.float32)
        copy(2 * j, rows0_v, sem0).wait()
        copyc(2 * j, cm0_v, semc0).wait()
        copy(2 * j + 1, rows1_v, sem1).start()
        copyc(2 * j + 1, cm1_v, semc1).start()
        acc = half(j, 0, rows0_v, cm0_v, (zero, zero))
        copy(2 * j + 1, rows1_v, sem1).wait()
        copyc(2 * j + 1, cm1_v, semc1).wait()

        @pl.when(j + 1 < NBATCH // 2)
        def _():
            copy(2 * j + 2, rows0_v, sem0).start()
            copyc(2 * j + 2, cm0_v, semc0).start()

        accA, accB = half(j, 1, rows1_v, cm1_v, acc)
        a_v[pl.ds(j * L, L)] = accA
        b_v[pl.ds(j * L, L)] = accB
        return 0

    lax.fori_loop(0, NBATCH // 2, pair_body, 0)
    pltpu.sync_copy(a_v, a_hbm.at[pl.ds(base, RPW)])
    pltpu.sync_copy(b_v, b_hbm.at[pl.ds(base, RPW)])


def _row_stats(d2, thr, cm):
    mesh = plsc.VectorSubcoreMesh(core_axis_name="c", subcore_axis_name="s")
    fn = functools.partial(
        pl.kernel, mesh=mesh,
        out_type=[jax.ShapeDtypeStruct((N,), jnp.float32),
                  jax.ShapeDtypeStruct((N,), jnp.float32)],
        scratch_types=[pltpu.VMEM((SCB, N), jnp.float32),
                       pltpu.VMEM((SCB, N), jnp.float32),
                       pltpu.VMEM((SCB, NCH), jnp.float32),
                       pltpu.VMEM((SCB, NCH), jnp.float32),
                       pltpu.VMEM((N + 2 * L,), jnp.float32),
                       pltpu.VMEM((RPW,), jnp.float32),
                       pltpu.VMEM((RPW,), jnp.float32),
                       pltpu.VMEM((RPW,), jnp.float32),
                       pltpu.SemaphoreType.DMA,
                       pltpu.SemaphoreType.DMA,
                       pltpu.SemaphoreType.DMA,
                       pltpu.SemaphoreType.DMA],
        compiler_params=pltpu.CompilerParams(needs_layout_passes=False),
    )(_stats_body)
    return fn(d2, thr, cm)


def _w_body(d2_ref, a_ref, b_ref, out_ref):
    i = pl.program_id(0)
    d2 = d2_ref[...]
    a_full = a_ref[...]
    b_full = b_ref[...]
    a_r = a_ref[pl.ds(i * RB, RB)]
    b_r = b_ref[pl.ds(i * RB, RB)]

    def inv_s2(a):
        s = jnp.maximum(jnp.sqrt(a + 1e-12), 1e-7)
        return 1.0 / (s * s)

    wr = jnp.where(d2 <= b_r[:, None],
                   jnp.exp(-d2 * inv_s2(a_r)[:, None]), 0.0)
    wc = jnp.where(d2 <= b_full[None, :],
                   jnp.exp(-d2 * inv_s2(a_full)[None, :]), 0.0)
    out_ref[...] = 0.5 * (wr + wc)


def _affinity_out(d2, a, b):
    return pl.pallas_call(
        _w_body,
        grid=(N // RB,),
        in_specs=[pl.BlockSpec((RB, N), lambda i: (i, 0)),
                  pl.BlockSpec((N,), lambda i: (0,)),
                  pl.BlockSpec((N,), lambda i: (0,))],
        out_specs=pl.BlockSpec((RB, N), lambda i: (i, 0)),
        out_shape=jax.ShapeDtypeStruct((N, N), jnp.float32),
    )(d2, a, b)


def kernel(X):
    d2, thr, cm = _pairwise_d2(X)
    a, b = _row_stats(d2, thr, cm)
    return _affinity_out(d2, a, b)


# speedup vs baseline: 1.2870x; 1.2870x over previous
import functools

import jax
import jax.numpy as jnp
from jax import lax
from jax.experimental import pallas as pl
from jax.experimental.pallas import tpu as pltpu
from jax.experimental.pallas import tpu_sc as plsc

N = 4096
DIM = 16
KSEL = 31
RB = 256
NBISECT = 10

NC = 2
NS = 16
L = 16
NW = NC * NS
RPW = N // NW
SCB = 8
NBATCH = RPW // SCB
NCH = 256
BIG = 1e30


def _d2_body(x_ref, out_ref, thr_ref):
    i = pl.program_id(0)
    X = x_ref[...]
    xb = x_ref[pl.ds(i * RB, RB), :]
    sq = jnp.sum(X * X, axis=1)
    sqb = jnp.sum(xb * xb, axis=1)
    dot = lax.dot_general(xb, X, (((1,), (1,)), ((), ())),
                          preferred_element_type=jnp.float32)
    d2 = jnp.maximum(sqb[:, None] + sq[None, :] - 2.0 * dot, 0.0)
    out_ref[...] = d2

    hi = (jnp.sqrt(sqb) + jnp.sqrt(jnp.max(sq))) ** 2 + 1.0
    lo = jnp.zeros((RB,), jnp.float32)
    for _ in range(NBISECT):
        mid = 0.5 * (lo + hi)
        c = jnp.sum((d2 < mid[:, None]).astype(jnp.int32), axis=1)
        ge = c >= KSEL
        hi = jnp.where(ge, mid, hi)
        lo = jnp.where(ge, lo, mid)
    thr_ref[...] = hi


def _pairwise_d2(X):
    return pl.pallas_call(
        _d2_body,
        grid=(N // RB,),
        in_specs=[pl.BlockSpec((N, DIM), lambda i: (0, 0))],
        out_specs=[pl.BlockSpec((RB, N), lambda i: (i, 0)),
                   pl.BlockSpec((RB,), lambda i: (i,))],
        out_shape=[jax.ShapeDtypeStruct((N, N), jnp.float32),
                   jax.ShapeDtypeStruct((N,), jnp.float32)],
    )(X)


def _merge32(A, B, v):
    vs = lax.sort(v)
    lo = jnp.minimum(B, lax.rev(vs, (0,)))
    rlos = lax.rev(lax.sort(lo), (0,))
    A2 = lax.sort(jnp.minimum(A, rlos))
    B2 = lax.sort(jnp.maximum(A, rlos))
    return A2, B2


def _filter_row(rows, r, t, cand_v):
    tv = jnp.broadcast_to(t, (L,))

    def group_b(g, off):
        base = g * (4 * L)
        vs = [rows[r, pl.ds(base + k * L, L)] for k in range(4)]
        mn = jnp.minimum(jnp.minimum(vs[0], vs[1]),
                         jnp.minimum(vs[2], vs[3]))

        def slow(off):
            for vk in vs:
                mask = vk < tv
                plsc.store_compressed(cand_v.at[pl.ds(off, L)], vk, mask=mask)
                off = off + plsc.all_reduce_population_count(mask)[0]
            return off

        has = plsc.all_reduce_population_count(mn < tv)[0] > 0
        return lax.cond(has, slow, lambda o: o, off)

    return lax.fori_loop(0, N // (4 * L), group_b, 0)


def _select_raw(rows, r):
    v0 = lax.sort(rows[r, pl.ds(0, L)])
    rv1 = lax.rev(lax.sort(rows[r, pl.ds(L, L)]), (0,))
    A = lax.sort(jnp.minimum(v0, rv1))
    B = lax.sort(jnp.maximum(v0, rv1))

    def cb(i, carry):
        return _merge32(carry[0], carry[1], rows[r, pl.ds(i * L, L)])

    A, B = lax.fori_loop(2, N // L, cb, (A, B))
    a15 = jnp.max(A)
    lane = lax.iota(jnp.int32, L)
    b14 = jnp.max(jnp.where(lane < L - 1, B, -1.0))
    return a15, b14


def _select_stats(cand_v, m):
    inf_v = jnp.full((L,), BIG, jnp.float32)
    cand_v[pl.ds(m, L)] = inf_v
    cand_v[pl.ds(m + L, L)] = inf_v
    v0 = lax.sort(cand_v[pl.ds(0, L)])
    rv1 = lax.rev(lax.sort(cand_v[pl.ds(L, L)]), (0,))
    A = lax.sort(jnp.minimum(v0, rv1))
    B = lax.sort(jnp.maximum(v0, rv1))

    def chunk_body(i, carry):
        v = cand_v[pl.ds(i * L, L)]
        return _merge32(carry[0], carry[1], v)

    nch = (m + L - 1) // L
    A, B = lax.fori_loop(2, nch, chunk_body, (A, B))
    a15 = jnp.max(A)
    lane = lax.iota(jnp.int32, L)
    b14 = jnp.max(jnp.where(lane < L - 1, B, -1.0))
    return a15, b14


def _stats_body(d2_hbm, thr_hbm, a_hbm, b_hbm,
                rows0_v, rows1_v, cand_v, thr_v, a_v, b_v, sem0, sem1):
    wid = lax.axis_index("s") * NC + lax.axis_index("c")
    base = wid * RPW
    pltpu.sync_copy(thr_hbm.at[pl.ds(base, RPW)], thr_v)

    def copy(bi, buf_ref, sem):
        return pltpu.make_async_copy(
            d2_hbm.at[pl.ds(base + bi * SCB, SCB), :], buf_ref, sem)

    copy(0, rows0_v, sem0).start()

    def half(j, half_idx, buf_ref, acc):
        accA, accB = acc
        lane = lax.iota(jnp.int32, L)
        tvec = thr_v[pl.ds(j * L, L)]

        def row_body(r, acc):
            accA, accB = acc
            li = half_idx * SCB + r
            t = jnp.max(jnp.where(lane == li, tvec, -1.0))
            m = _filter_row(buf_ref, r, t, cand_v)
            a15, b14 = lax.cond(m >= KSEL,
                                lambda: _select_stats(cand_v, m),
                                lambda: _select_raw(buf_ref, r))
            accA = jnp.where(lane == li, a15, accA)
            accB = jnp.where(lane == li, b14, accB)
            return accA, accB

        return lax.fori_loop(0, SCB, row_body, (accA, accB))

    def pair_body(j, _):
        zero = jnp.zeros((L,), jnp.float32)
        copy(2 * j, rows0_v, sem0).wait()
        copy(2 * j + 1, rows1_v, sem1).start()
        acc = half(j, 0, rows0_v, (zero, zero))
        copy(2 * j + 1, rows1_v, sem1).wait()

        @pl.when(j + 1 < NBATCH // 2)
        def _():
            copy(2 * j + 2, rows0_v, sem0).start()

        accA, accB = half(j, 1, rows1_v, acc)
        a_v[pl.ds(j * L, L)] = accA
        b_v[pl.ds(j * L, L)] = accB
        return 0

    lax.fori_loop(0, NBATCH // 2, pair_body, 0)
    pltpu.sync_copy(a_v, a_hbm.at[pl.ds(base, RPW)])
    pltpu.sync_copy(b_v, b_hbm.at[pl.ds(base, RPW)])


def _row_stats(d2, thr):
    mesh = plsc.VectorSubcoreMesh(core_axis_name="c", subcore_axis_name="s")
    fn = functools.partial(
        pl.kernel, mesh=mesh,
        out_type=[jax.ShapeDtypeStruct((N,), jnp.float32),
                  jax.ShapeDtypeStruct((N,), jnp.float32)],
        scratch_types=[pltpu.VMEM((SCB, N), jnp.float32),
                       pltpu.VMEM((SCB, N), jnp.float32),
                       pltpu.VMEM((N + 2 * L,), jnp.float32),
                       pltpu.VMEM((RPW,), jnp.float32),
                       pltpu.VMEM((RPW,), jnp.float32),
                       pltpu.VMEM((RPW,), jnp.float32),
                       pltpu.SemaphoreType.DMA,
                       pltpu.SemaphoreType.DMA],
        compiler_params=pltpu.CompilerParams(needs_layout_passes=False),
    )(_stats_body)
    return fn(d2, thr)


def _w_body(d2_ref, a_ref, b_ref, out_ref):
    i = pl.program_id(0)
    d2 = d2_ref[...]
    a_full = a_ref[...]
    b_full = b_ref[...]
    a_r = a_ref[pl.ds(i * RB, RB)]
    b_r = b_ref[pl.ds(i * RB, RB)]

    def inv_s2(a):
        s = jnp.maximum(jnp.sqrt(a + 1e-12), 1e-7)
        return 1.0 / (s * s)

    wr = jnp.where(d2 <= b_r[:, None],
                   jnp.exp(-d2 * inv_s2(a_r)[:, None]), 0.0)
    wc = jnp.where(d2 <= b_full[None, :],
                   jnp.exp(-d2 * inv_s2(a_full)[None, :]), 0.0)
    out_ref[...] = 0.5 * (wr + wc)


def _affinity_out(d2, a, b):
    return pl.pallas_call(
        _w_body,
        grid=(N // RB,),
        in_specs=[pl.BlockSpec((RB, N), lambda i: (i, 0)),
                  pl.BlockSpec((N,), lambda i: (0,)),
                  pl.BlockSpec((N,), lambda i: (0,))],
        out_specs=pl.BlockSpec((RB, N), lambda i: (i, 0)),
        out_shape=jax.ShapeDtypeStruct((N, N), jnp.float32),
    )(d2, a, b)


def kernel(X):
    d2, thr = _pairwise_d2(X)
    a, b = _row_stats(d2, thr)
    return _affinity_out(d2, a, b)


# speedup vs baseline: 1.3217x; 1.0270x over previous
import functools

import jax
import jax.numpy as jnp
from jax import lax
from jax.experimental import pallas as pl
from jax.experimental.pallas import tpu as pltpu
from jax.experimental.pallas import tpu_sc as plsc

N = 4096
DIM = 16
KSEL = 31
RB = 256
NBISECT = 8

NC = 2
NS = 16
L = 16
NW = NC * NS
RPW = N // NW
SCB = 8
NBATCH = RPW // SCB
NCH = 256
BIG = 1e30


def _d2_body(x_ref, out_ref, thr_ref):
    i = pl.program_id(0)
    X = x_ref[...]
    xb = x_ref[pl.ds(i * RB, RB), :]
    sq = jnp.sum(X * X, axis=1)
    sqb = jnp.sum(xb * xb, axis=1)
    dot = lax.dot_general(xb, X, (((1,), (1,)), ((), ())),
                          preferred_element_type=jnp.float32)
    d2 = jnp.maximum(sqb[:, None] + sq[None, :] - 2.0 * dot, 0.0)
    out_ref[...] = d2

    hi = (jnp.sqrt(sqb) + jnp.sqrt(jnp.max(sq))) ** 2 + 1.0
    lo = jnp.zeros((RB,), jnp.float32)
    for _ in range(NBISECT):
        mid = 0.5 * (lo + hi)
        c = jnp.sum((d2 < mid[:, None]).astype(jnp.int32), axis=1)
        ge = c >= KSEL
        hi = jnp.where(ge, mid, hi)
        lo = jnp.where(ge, lo, mid)
    thr_ref[...] = hi


def _pairwise_d2(X):
    return pl.pallas_call(
        _d2_body,
        grid=(N // RB,),
        in_specs=[pl.BlockSpec((N, DIM), lambda i: (0, 0))],
        out_specs=[pl.BlockSpec((RB, N), lambda i: (i, 0)),
                   pl.BlockSpec((RB,), lambda i: (i,))],
        out_shape=[jax.ShapeDtypeStruct((N, N), jnp.float32),
                   jax.ShapeDtypeStruct((N,), jnp.float32)],
    )(X)


def _merge32(A, B, v):
    vs = lax.sort(v)
    lo = jnp.minimum(B, lax.rev(vs, (0,)))
    rlos = lax.rev(lax.sort(lo), (0,))
    A2 = lax.sort(jnp.minimum(A, rlos))
    B2 = lax.sort(jnp.maximum(A, rlos))
    return A2, B2


def _filter_row(rows, r, t, cand_v):
    tv = jnp.broadcast_to(t, (L,))

    def group_b(g, off):
        base = g * (4 * L)
        vs = [rows[r, pl.ds(base + k * L, L)] for k in range(4)]
        mn = jnp.minimum(jnp.minimum(vs[0], vs[1]),
                         jnp.minimum(vs[2], vs[3]))

        def slow(off):
            for vk in vs:
                mask = vk < tv
                plsc.store_compressed(cand_v.at[pl.ds(off, L)], vk, mask=mask)
                off = off + plsc.all_reduce_population_count(mask)[0]
            return off

        has = plsc.all_reduce_population_count(mn < tv)[0] > 0
        return lax.cond(has, slow, lambda o: o, off)

    return lax.fori_loop(0, N // (4 * L), group_b, 0)


def _select_raw(rows, r):
    v0 = lax.sort(rows[r, pl.ds(0, L)])
    rv1 = lax.rev(lax.sort(rows[r, pl.ds(L, L)]), (0,))
    A = lax.sort(jnp.minimum(v0, rv1))
    B = lax.sort(jnp.maximum(v0, rv1))

    def cb(i, carry):
        return _merge32(carry[0], carry[1], rows[r, pl.ds(i * L, L)])

    A, B = lax.fori_loop(2, N // L, cb, (A, B))
    a15 = jnp.max(A)
    lane = lax.iota(jnp.int32, L)
    b14 = jnp.max(jnp.where(lane < L - 1, B, -1.0))
    return a15, b14


def _select_stats(cand_v, m):
    inf_v = jnp.full((L,), BIG, jnp.float32)
    cand_v[pl.ds(m, L)] = inf_v
    cand_v[pl.ds(m + L, L)] = inf_v
    v0 = lax.sort(cand_v[pl.ds(0, L)])
    rv1 = lax.rev(lax.sort(cand_v[pl.ds(L, L)]), (0,))
    A = lax.sort(jnp.minimum(v0, rv1))
    B = lax.sort(jnp.maximum(v0, rv1))

    def chunk_body(i, carry):
        v = cand_v[pl.ds(i * L, L)]
        return _merge32(carry[0], carry[1], v)

    nch = (m + L - 1) // L
    A, B = lax.fori_loop(2, nch, chunk_body, (A, B))
    a15 = jnp.max(A)
    lane = lax.iota(jnp.int32, L)
    b14 = jnp.max(jnp.where(lane < L - 1, B, -1.0))
    return a15, b14


def _stats_body(d2_hbm, thr_hbm, a_hbm, b_hbm,
                rows0_v, rows1_v, cand_v, thr_v, a_v, b_v, sem0, sem1):
    wid = lax.axis_index("s") * NC + lax.axis_index("c")
    base = wid * RPW
    pltpu.sync_copy(thr_hbm.at[pl.ds(base, RPW)], thr_v)

    def copy(bi, buf_ref, sem):
        return pltpu.make_async_copy(
            d2_hbm.at[pl.ds(base + bi * SCB, SCB), :], buf_ref, sem)

    copy(0, rows0_v, sem0).start()

    def half(j, half_idx, buf_ref, acc):
        accA, accB = acc
        lane = lax.iota(jnp.int32, L)
        tvec = thr_v[pl.ds(j * L, L)]

        def row_body(r, acc):
            accA, accB = acc
            li = half_idx * SCB + r
            t = jnp.max(jnp.where(lane == li, tvec, -1.0))
            m = _filter_row(buf_ref, r, t, cand_v)
            a15, b14 = lax.cond(m >= KSEL,
                                lambda: _select_stats(cand_v, m),
                                lambda: _select_raw(buf_ref, r))
            accA = jnp.where(lane == li, a15, accA)
            accB = jnp.where(lane == li, b14, accB)
            return accA, accB

        return lax.fori_loop(0, SCB, row_body, (accA, accB))

    def pair_body(j, _):
        zero = jnp.zeros((L,), jnp.float32)
        copy(2 * j, rows0_v, sem0).wait()
        copy(2 * j + 1, rows1_v, sem1).start()
        acc = half(j, 0, rows0_v, (zero, zero))
        copy(2 * j + 1, rows1_v, sem1).wait()

        @pl.when(j + 1 < NBATCH // 2)
        def _():
            copy(2 * j + 2, rows0_v, sem0).start()

        accA, accB = half(j, 1, rows1_v, acc)
        a_v[pl.ds(j * L, L)] = accA
        b_v[pl.ds(j * L, L)] = accB
        return 0

    lax.fori_loop(0, NBATCH // 2, pair_body, 0)
    pltpu.sync_copy(a_v, a_hbm.at[pl.ds(base, RPW)])
    pltpu.sync_copy(b_v, b_hbm.at[pl.ds(base, RPW)])


def _row_stats(d2, thr):
    mesh = plsc.VectorSubcoreMesh(core_axis_name="c", subcore_axis_name="s")
    fn = functools.partial(
        pl.kernel, mesh=mesh,
        out_type=[jax.ShapeDtypeStruct((N,), jnp.float32),
                  jax.ShapeDtypeStruct((N,), jnp.float32)],
        scratch_types=[pltpu.VMEM((SCB, N), jnp.float32),
                       pltpu.VMEM((SCB, N), jnp.float32),
                       pltpu.VMEM((N + 2 * L,), jnp.float32),
                       pltpu.VMEM((RPW,), jnp.float32),
                       pltpu.VMEM((RPW,), jnp.float32),
                       pltpu.VMEM((RPW,), jnp.float32),
                       pltpu.SemaphoreType.DMA,
                       pltpu.SemaphoreType.DMA],
        compiler_params=pltpu.CompilerParams(needs_layout_passes=False),
    )(_stats_body)
    return fn(d2, thr)


def _w_body(d2_ref, a_ref, b_ref, out_ref):
    i = pl.program_id(0)
    d2 = d2_ref[...]
    a_full = a_ref[...]
    b_full = b_ref[...]
    a_r = a_ref[pl.ds(i * RB, RB)]
    b_r = b_ref[pl.ds(i * RB, RB)]

    def inv_s2(a):
        s = jnp.maximum(jnp.sqrt(a + 1e-12), 1e-7)
        return 1.0 / (s * s)

    wr = jnp.where(d2 <= b_r[:, None],
                   jnp.exp(-d2 * inv_s2(a_r)[:, None]), 0.0)
    wc = jnp.where(d2 <= b_full[None, :],
                   jnp.exp(-d2 * inv_s2(a_full)[None, :]), 0.0)
    out_ref[...] = 0.5 * (wr + wc)


def _affinity_out(d2, a, b):
    return pl.pallas_call(
        _w_body,
        grid=(N // RB,),
        in_specs=[pl.BlockSpec((RB, N), lambda i: (i, 0)),
                  pl.BlockSpec((N,), lambda i: (0,)),
                  pl.BlockSpec((N,), lambda i: (0,))],
        out_specs=pl.BlockSpec((RB, N), lambda i: (i, 0)),
        out_shape=jax.ShapeDtypeStruct((N, N), jnp.float32),
    )(d2, a, b)


def kernel(X):
    d2, thr = _pairwise_d2(X)
    a, b = _row_stats(d2, thr)
    return _affinity_out(d2, a, b)
